# baseline (speedup 1.0000x reference)
import functools

import jax
import jax.numpy as jnp
from jax import lax
from jax.experimental import pallas as pl
from jax.experimental.pallas import tpu as pltpu
from jax.experimental.pallas import tpu_sc as plsc

_NC = 2
_NS = 16
_NW = _NC * _NS
_CHUNK = 128
_LANES = 16
_TILE_W = 128
_DEPTH = 6


def _make_item_gather(B, FI, bpw, k):
  mesh = plsc.VectorSubcoreMesh(core_axis_name="c", subcore_axis_name="s")

  @functools.partial(
      pl.kernel,
      mesh=mesh,
      out_type=jax.ShapeDtypeStruct((B, FI), jnp.float32),
      scratch_types=[
          pltpu.VMEM((k, _CHUNK), jnp.int32),
          pltpu.VMEM((bpw, FI), jnp.float32),
          pltpu.SemaphoreType.DMA,
      ],
      compiler_params=pltpu.CompilerParams(needs_layout_passes=False),
  )
  def gather(feat_hbm, nidi_hbm, xi_hbm, idxi_v, rowsi_v, sem):
    wid = lax.axis_index("s") * _NC + lax.axis_index("c")
    base = wid * bpw
    pltpu.sync_copy(nidi_hbm.at[wid], idxi_v)
    copies = [
        pltpu.async_copy(
            feat_hbm.at[idxi_v.at[j]], rowsi_v.at[pl.ds(j * _CHUNK, _CHUNK)],
            sem)
        for j in range(k)
    ]
    for c in copies:
      c.wait()
    pltpu.sync_copy(rowsi_v, xi_hbm.at[pl.ds(base, bpw)])

  return gather


def _make_user_gather(R, EU, B):
  mesh = plsc.VectorSubcoreMesh(core_axis_name="c", subcore_axis_name="s")
  n_full = (R // _TILE_W) * _TILE_W
  tail_w = R - n_full
  nslab = R // _TILE_W + (1 if tail_w else 0)
  eq = EU // _LANES
  max_bins = (nslab + _NW - 1) // _NW + 1
  bins_pad = ((max_bins + 15) // 16) * 16
  n_outer = (max_bins + _DEPTH - 1) // _DEPTH
  fb = 16
  dump = B

  @functools.partial(
      pl.kernel,
      mesh=mesh,
      out_type=jax.ShapeDtypeStruct((B + 8, _TILE_W), jnp.float32),
      scratch_types=[
          pltpu.VMEM((B + 16,), jnp.int32),
          pltpu.VMEM((B + 16,), jnp.int32),
          pltpu.VMEM((B + 16,), jnp.int32),
          pltpu.VMEM((B + 32,), jnp.int32),
          pltpu.VMEM((bins_pad,), jnp.int32),
          pltpu.VMEM((bins_pad,), jnp.int32),
          [pltpu.VMEM((EU, _TILE_W), jnp.float32) for _ in range(_DEPTH)],
          pltpu.VMEM((EU, tail_w), jnp.float32) if tail_w else None,
          pltpu.VMEM((2 * fb, _TILE_W), jnp.float32),
          pltpu.VMEM((2, fb), jnp.int32),
          [pltpu.SemaphoreType.DMA for _ in range(_DEPTH)],
          [pltpu.SemaphoreType.DMA for _ in range(_DEPTH)],
          pltpu.SemaphoreType.DMA,
      ],
      compiler_params=pltpu.CompilerParams(needs_layout_passes=False),
  )
  def gather(embt_hbm, nidu_hbm, out2_hbm, idx_all, hit_r, hit_p, sorted_p,
             counts, starts, slabs, tail_v, staging, posb, sems, sems2, semf):
    wid = lax.axis_index("s") * _NC + lax.axis_index("c")
    lo = (wid * nslab) // _NW
    hi = ((wid + 1) * nslab) // _NW
    lane = lax.iota(jnp.int32, _LANES)
    ones = jnp.ones((_LANES,), jnp.int32)
    zeros = jnp.zeros((_LANES,), jnp.int32)
    sorted_r = idx_all

    pltpu.sync_copy(nidu_hbm, idx_all.at[pl.ds(0, B)])
    if tail_w:
      pltpu.sync_copy(embt_hbm.at[:, pl.ds(n_full, tail_w)], tail_v)

    def scan(t, n):
      v = idx_all[pl.ds(t * _LANES, _LANES)]
      b16 = v // _TILE_W
      m = (b16 >= lo) & (b16 < hi)
      np_ = plsc.all_reduce_population_count(m)[0]

      @pl.when(np_ > 0)
      def _():
        plsc.store_compressed(hit_r.at[pl.ds(n, _LANES)], v, mask=m)
        plsc.store_compressed(
            hit_p.at[pl.ds(n, _LANES)], lane + t * _LANES, mask=m)
      return n + np_

    n = lax.fori_loop(0, B // _LANES, scan, 0)

    for q in range(bins_pad // 16):
      counts[pl.ds(q * 16, 16)] = zeros

    def hist(t, _):
      v = hit_r[pl.ds(t * _LANES, _LANES)]
      valid = (lane + t * _LANES) < n
      bl = jnp.where(valid, v // _TILE_W - lo, 0)
      plsc.addupdate_scatter(counts, [bl], ones, mask=valid)
      return ()

    lax.fori_loop(0, (n + _LANES - 1) // _LANES, hist, ())

    def prefix(q, run):
      c = counts[pl.ds(q * 16, 16)]
      cum = plsc.cumsum(c)
      starts[pl.ds(q * 16, 16)] = cum - c + run
      return run + jnp.sum(c)

    lax.fori_loop(0, bins_pad // 16, prefix, 0)

    def place(k, _):
      cb = (k // _LANES) * _LANES
      l = k % _LANES
      vr = hit_r[pl.ds(cb, _LANES)]
      vp = hit_p[pl.ds(cb, _LANES)]
      r = jnp.sum(jnp.where(lane == l, vr, 0))
      p = jnp.sum(jnp.where(lane == l, vp, 0))
      bl16 = zeros + (r // _TILE_W - lo)
      o16 = plsc.load_gather(starts, [bl16])
      m0 = lane == 0
      plsc.store_scatter(sorted_r, [o16], zeros + r, mask=m0)
      plsc.store_scatter(sorted_p, [o16], zeros + p, mask=m0)
      plsc.addupdate_scatter(starts, [bl16], ones, mask=m0)
      return ()

    lax.fori_loop(0, n, place, ())

    def bin_cnt(mi):
      c16 = plsc.load_gather(counts, [zeros + mi])
      return jnp.sum(jnp.where(lane == 0, c16, 0))

    def bin_end(mi):
      e16 = plsc.load_gather(starts, [zeros + mi])
      return jnp.sum(jnp.where(lane == 0, e16, 0))

    n_fullbins = (R // _TILE_W)

    def issue(mi, slot, sem, sem2):
      @pl.when((mi < hi - lo) & (lo + mi < n_fullbins) & (bin_cnt(mi) > 0))
      def _():
        off = pl.multiple_of((lo + mi) * _TILE_W, _TILE_W)
        h = EU // 2
        pltpu.async_copy(
            embt_hbm.at[pl.ds(0, h), pl.ds(off, _TILE_W)],
            slot.at[pl.ds(0, h)], sem)
        pltpu.async_copy(
            embt_hbm.at[pl.ds(h, h), pl.ds(off, _TILE_W)],
            slot.at[pl.ds(h, h)], sem2)

    def drain(mi, slot, sem, sem2):
      @pl.when((mi < hi - lo) & (lo + mi < n_fullbins) & (bin_cnt(mi) > 0))
      def _():
        h = EU // 2
        pltpu.make_async_copy(
            embt_hbm.at[pl.ds(0, h), pl.ds(0, _TILE_W)],
            slot.at[pl.ds(0, h)], sem).wait()
        pltpu.make_async_copy(
            embt_hbm.at[pl.ds(h, h), pl.ds(0, _TILE_W)],
            slot.at[pl.ds(h, h)], sem2).wait()

    def drain_flush(fc):
      pb = fc % 2
      src = staging.at[pl.ds(pl.multiple_of(pb * fb, fb), fb)]
      pltpu.make_async_copy(src, out2_hbm.at[posb.at[pb]], semf).wait()

    def flush(fc, rem):
      pb = fc % 2
      for u in range(fb // 16):
        pc = sorted_p[pl.ds(fc * fb + u * 16, 16)]
        valid = (u * 16 + lane) < rem
        posb[pb, pl.ds(u * 16, 16)] = jnp.where(
            valid, pc, dump + (wid % 8))
      src = staging.at[pl.ds(pl.multiple_of(pb * fb, fb), fb)]
      pltpu.async_copy(src, out2_hbm.at[posb.at[pb]], semf)

    def extract_bin(mi, slot, g):
      cnt = bin_cnt(mi)
      begin = bin_end(mi) - cnt
      from_tail = (lo + mi == nslab - 1) if tail_w else False

      def one(k, g):
        kk = begin + k
        cb = (kk // _LANES) * _LANES
        l = kk % _LANES
        @pl.when((g % fb == 0) & (g >= 2 * fb))
        def _():
          drain_flush(g // fb - 2)
        vr = sorted_r[pl.ds(cb, _LANES)]
        r = jnp.sum(jnp.where(lane == l, vr, 0))
        cs = zeros + (r % _TILE_W)
        srow = g % (2 * fb)
        for q in range(eq):
          e16 = lane + q * _LANES
          if tail_w:
            gv = jnp.where(
                from_tail,
                plsc.load_gather(
                    tail_v, [e16, jnp.minimum(cs, tail_w - 1)]),
                plsc.load_gather(slot, [e16, cs]))
          else:
            gv = plsc.load_gather(slot, [e16, cs])
          staging[srow, pl.ds(q * _LANES, _LANES)] = gv

        @pl.when((g + 1) % fb == 0)
        def _():
          flush((g + 1) // fb - 1, fb)
        return g + 1

      return lax.fori_loop(0, cnt, one, g)

    for d in range(_DEPTH):
      issue(d, slabs[d], sems[d], sems2[d])

    def outer(o, g):
      for d in range(_DEPTH):
        mi = o * _DEPTH + d
        drain(mi, slabs[d], sems[d], sems2[d])
        g = extract_bin(mi, slabs[d], g)
        issue(mi + _DEPTH, slabs[d], sems[d], sems2[d])
      return g

    g = lax.fori_loop(0, n_outer, outer, 0)

    @pl.when(g % fb != 0)
    def _():
      flush(g // fb, g % fb)

    nf = (g + fb - 1) // fb

    @pl.when(nf >= 2)
    def _():
      drain_flush(nf - 2)

    @pl.when(nf >= 1)
    def _():
      drain_flush(nf - 1)

  return gather


def _proj_body(x_ref, w_ref, o_ref):
  o_ref[...] = lax.dot_general(
      w_ref[...], x_ref[...],
      dimension_numbers=(((0,), (1,)), ((), ())),
      preferred_element_type=jnp.float32)


@jax.jit
def kernel(nid_user, nid_item, emb_user, feat_item, W_item):
  B = nid_user.shape[0]
  R = emb_user.shape[0]
  EU = emb_user.shape[1]
  FI = feat_item.shape[1]
  EO = W_item.shape[1]
  bpw = B // _NW
  k = bpw // _CHUNK

  nidu = nid_user.astype(jnp.int32)
  nidi = nid_item.astype(jnp.int32).reshape(_NW, k, _CHUNK)

  item_rows = _make_item_gather(B, FI, bpw, k)(feat_item, nidi)
  xu2 = _make_user_gather(R, EU, B)(emb_user.T, nidu)
  x_user = xu2[:B, :EU]

  rows_blk = 2048
  xit = pl.pallas_call(
      _proj_body,
      grid=(B // rows_blk,),
      in_specs=[
          pl.BlockSpec((rows_blk, FI), lambda i: (i, 0)),
          pl.BlockSpec((FI, EO), lambda i: (0, 0)),
      ],
      out_specs=pl.BlockSpec((EO, rows_blk), lambda i: (0, i)),
      out_shape=jax.ShapeDtypeStruct((EO, B), jnp.float32),
  )(item_rows, W_item)

  return (x_user, xit.T)


# speedup vs baseline: 1.0276x; 1.0276x over previous
import functools

import jax
import jax.numpy as jnp
from jax import lax
from jax.experimental import pallas as pl
from jax.experimental.pallas import tpu as pltpu
from jax.experimental.pallas import tpu_sc as plsc

_NC = 2
_NS = 16
_NW = _NC * _NS
_CHUNK = 128
_LANES = 16
_TILE_W = 128
_DEPTH = 6


def _make_item_gather(B, FI, bpw, k):
  mesh = plsc.VectorSubcoreMesh(core_axis_name="c", subcore_axis_name="s")

  @functools.partial(
      pl.kernel,
      mesh=mesh,
      out_type=jax.ShapeDtypeStruct((B, FI), jnp.float32),
      scratch_types=[
          pltpu.VMEM((k, _CHUNK), jnp.int32),
          pltpu.VMEM((bpw, FI), jnp.float32),
          pltpu.SemaphoreType.DMA,
      ],
      compiler_params=pltpu.CompilerParams(needs_layout_passes=False),
  )
  def gather(feat_hbm, nidi_hbm, xi_hbm, idxi_v, rowsi_v, sem):
    wid = lax.axis_index("s") * _NC + lax.axis_index("c")
    base = wid * bpw
    pltpu.sync_copy(nidi_hbm.at[wid], idxi_v)
    copies = [
        pltpu.async_copy(
            feat_hbm.at[idxi_v.at[j]], rowsi_v.at[pl.ds(j * _CHUNK, _CHUNK)],
            sem)
        for j in range(k)
    ]
    for c in copies:
      c.wait()
    pltpu.sync_copy(rowsi_v, xi_hbm.at[pl.ds(base, bpw)])

  return gather


def _make_user_gather(R, EU, B):
  mesh = plsc.VectorSubcoreMesh(core_axis_name="c", subcore_axis_name="s")
  n_full = (R // _TILE_W) * _TILE_W
  tail_w = R - n_full
  nslab = R // _TILE_W + (1 if tail_w else 0)
  eq = EU // _LANES
  max_bins = (nslab + _NW - 1) // _NW + 1
  bins_pad = ((max_bins + 15) // 16) * 16
  n_outer = (max_bins + _DEPTH - 1) // _DEPTH
  fb = 16
  dump = B

  @functools.partial(
      pl.kernel,
      mesh=mesh,
      out_type=jax.ShapeDtypeStruct((B + 8, _TILE_W), jnp.float32),
      scratch_types=[
          pltpu.VMEM((B + 16,), jnp.int32),
          pltpu.VMEM((B + 16,), jnp.int32),
          pltpu.VMEM((B + 16,), jnp.int32),
          pltpu.VMEM((B + 32,), jnp.int32),
          pltpu.VMEM((bins_pad,), jnp.int32),
          pltpu.VMEM((bins_pad,), jnp.int32),
          [pltpu.VMEM((EU, _TILE_W), jnp.float32) for _ in range(_DEPTH)],
          pltpu.VMEM((EU, tail_w), jnp.float32) if tail_w else None,
          pltpu.VMEM((2 * fb, _TILE_W), jnp.float32),
          pltpu.VMEM((2, fb), jnp.int32),
          [pltpu.SemaphoreType.DMA for _ in range(_DEPTH)],
          pltpu.SemaphoreType.DMA,
      ],
      compiler_params=pltpu.CompilerParams(needs_layout_passes=False),
  )
  def gather(embt_hbm, nidu_hbm, out2_hbm, idx_all, hit_r, hit_p, sorted_p,
             counts, starts, slabs, tail_v, staging, posb, sems, semf):
    wid = lax.axis_index("s") * _NC + lax.axis_index("c")
    lo = (wid * nslab) // _NW
    hi = ((wid + 1) * nslab) // _NW
    lane = lax.iota(jnp.int32, _LANES)
    ones = jnp.ones((_LANES,), jnp.int32)
    zeros = jnp.zeros((_LANES,), jnp.int32)
    sorted_r = idx_all

    pltpu.sync_copy(nidu_hbm, idx_all.at[pl.ds(0, B)])
    if tail_w:
      pltpu.sync_copy(embt_hbm.at[:, pl.ds(n_full, tail_w)], tail_v)

    def scan(t, n):
      v = idx_all[pl.ds(t * _LANES, _LANES)]
      b16 = v // _TILE_W
      m = (b16 >= lo) & (b16 < hi)
      np_ = plsc.all_reduce_population_count(m)[0]

      @pl.when(np_ > 0)
      def _():
        plsc.store_compressed(hit_r.at[pl.ds(n, _LANES)], v, mask=m)
        plsc.store_compressed(
            hit_p.at[pl.ds(n, _LANES)], lane + t * _LANES, mask=m)
      return n + np_

    n = lax.fori_loop(0, B // _LANES, scan, 0)

    for q in range(bins_pad // 16):
      counts[pl.ds(q * 16, 16)] = zeros

    def hist(t, _):
      v = hit_r[pl.ds(t * _LANES, _LANES)]
      valid = (lane + t * _LANES) < n
      bl = jnp.where(valid, v // _TILE_W - lo, 0)
      plsc.addupdate_scatter(counts, [bl], ones, mask=valid)
      return ()

    lax.fori_loop(0, (n + _LANES - 1) // _LANES, hist, ())

    def prefix(q, run):
      c = counts[pl.ds(q * 16, 16)]
      cum = plsc.cumsum(c)
      starts[pl.ds(q * 16, 16)] = cum - c + run
      return run + jnp.sum(c)

    lax.fori_loop(0, bins_pad // 16, prefix, 0)

    def place(k, _):
      cb = (k // _LANES) * _LANES
      l = k % _LANES
      vr = hit_r[pl.ds(cb, _LANES)]
      vp = hit_p[pl.ds(cb, _LANES)]
      r = jnp.sum(jnp.where(lane == l, vr, 0))
      p = jnp.sum(jnp.where(lane == l, vp, 0))
      bl16 = zeros + (r // _TILE_W - lo)
      o16 = plsc.load_gather(starts, [bl16])
      m0 = lane == 0
      plsc.store_scatter(sorted_r, [o16], zeros + r, mask=m0)
      plsc.store_scatter(sorted_p, [o16], zeros + p, mask=m0)
      plsc.addupdate_scatter(starts, [bl16], ones, mask=m0)
      return ()

    lax.fori_loop(0, n, place, ())

    def bin_cnt(mi):
      c16 = plsc.load_gather(counts, [zeros + mi])
      return jnp.sum(jnp.where(lane == 0, c16, 0))

    def bin_end(mi):
      e16 = plsc.load_gather(starts, [zeros + mi])
      return jnp.sum(jnp.where(lane == 0, e16, 0))

    n_fullbins = (R // _TILE_W)

    def issue(mi, slot, sem):
      @pl.when((mi < hi - lo) & (lo + mi < n_fullbins) & (bin_cnt(mi) > 0))
      def _():
        off = pl.multiple_of((lo + mi) * _TILE_W, _TILE_W)
        pltpu.async_copy(embt_hbm.at[:, pl.ds(off, _TILE_W)], slot, sem)

    def drain(mi, slot, sem):
      @pl.when((mi < hi - lo) & (lo + mi < n_fullbins) & (bin_cnt(mi) > 0))
      def _():
        pltpu.make_async_copy(
            embt_hbm.at[:, pl.ds(0, _TILE_W)], slot, sem).wait()

    def drain_flush(fc):
      pb = fc % 2
      src = staging.at[pl.ds(pl.multiple_of(pb * fb, fb), fb)]
      pltpu.make_async_copy(src, out2_hbm.at[posb.at[pb]], semf).wait()

    def flush(fc, rem):
      pb = fc % 2
      for u in range(fb // 16):
        pc = sorted_p[pl.ds(fc * fb + u * 16, 16)]
        valid = (u * 16 + lane) < rem
        posb[pb, pl.ds(u * 16, 16)] = jnp.where(
            valid, pc, dump + (wid % 8))
      src = staging.at[pl.ds(pl.multiple_of(pb * fb, fb), fb)]
      pltpu.async_copy(src, out2_hbm.at[posb.at[pb]], semf)

    def extract_bin(mi, slot, g):
      cnt = bin_cnt(mi)
      begin = bin_end(mi) - cnt
      from_tail = (lo + mi == nslab - 1) if tail_w else False

      def one(k, g):
        kk = begin + k
        cb = (kk // _LANES) * _LANES
        l = kk % _LANES
        @pl.when((g % fb == 0) & (g >= 2 * fb))
        def _():
          drain_flush(g // fb - 2)
        vr = sorted_r[pl.ds(cb, _LANES)]
        r = jnp.sum(jnp.where(lane == l, vr, 0))
        cs = zeros + (r % _TILE_W)
        srow = g % (2 * fb)
        for q in range(eq):
          e16 = lane + q * _LANES
          if tail_w:
            gv = jnp.where(
                from_tail,
                plsc.load_gather(
                    tail_v, [e16, jnp.minimum(cs, tail_w - 1)]),
                plsc.load_gather(slot, [e16, cs]))
          else:
            gv = plsc.load_gather(slot, [e16, cs])
          staging[srow, pl.ds(q * _LANES, _LANES)] = gv

        @pl.when((g + 1) % fb == 0)
        def _():
          flush((g + 1) // fb - 1, fb)
        return g + 1

      return lax.fori_loop(0, cnt, one, g)

    for d in range(_DEPTH):
      issue(d, slabs[d], sems[d])

    def outer(o, g):
      for d in range(_DEPTH):
        mi = o * _DEPTH + d
        drain(mi, slabs[d], sems[d])
        g = extract_bin(mi, slabs[d], g)
        issue(mi + _DEPTH, slabs[d], sems[d])
      return g

    g = lax.fori_loop(0, n_outer, outer, 0)

    @pl.when(g % fb != 0)
    def _():
      flush(g // fb, g % fb)

    nf = (g + fb - 1) // fb

    @pl.when(nf >= 2)
    def _():
      drain_flush(nf - 2)

    @pl.when(nf >= 1)
    def _():
      drain_flush(nf - 1)

  return gather


def _proj_body(x_ref, u_ref, w_ref, o_ref, ou_ref):
  o_ref[...] = lax.dot_general(
      w_ref[...], x_ref[...],
      dimension_numbers=(((0,), (1,)), ((), ())),
      preferred_element_type=jnp.float32)
  eu = ou_ref.shape[0]
  row = lax.broadcasted_iota(jnp.int32, (eu, eu), 0)
  col = lax.broadcasted_iota(jnp.int32, (eu, eu), 1)
  eye = jnp.where(row == col, 1.0, 0.0).astype(jnp.float32)
  ou_ref[...] = lax.dot_general(
      eye, u_ref[pl.ds(0, u_ref.shape[0]), pl.ds(0, eu)],
      dimension_numbers=(((1,), (1,)), ((), ())),
      preferred_element_type=jnp.float32)


@jax.jit
def kernel(nid_user, nid_item, emb_user, feat_item, W_item):
  B = nid_user.shape[0]
  R = emb_user.shape[0]
  EU = emb_user.shape[1]
  FI = feat_item.shape[1]
  EO = W_item.shape[1]
  bpw = B // _NW
  k = bpw // _CHUNK

  nidu = nid_user.astype(jnp.int32)
  nidi = nid_item.astype(jnp.int32).reshape(_NW, k, _CHUNK)

  item_rows = _make_item_gather(B, FI, bpw, k)(feat_item, nidi)
  xu2 = _make_user_gather(R, EU, B)(emb_user.T, nidu)

  rows_blk = 2048
  xit, xut = pl.pallas_call(
      _proj_body,
      grid=(B // rows_blk,),
      in_specs=[
          pl.BlockSpec((rows_blk, FI), lambda i: (i, 0)),
          pl.BlockSpec((rows_blk, _TILE_W), lambda i: (i, 0)),
          pl.BlockSpec((FI, EO), lambda i: (0, 0)),
      ],
      out_specs=[
          pl.BlockSpec((EO, rows_blk), lambda i: (0, i)),
          pl.BlockSpec((EU, rows_blk), lambda i: (0, i)),
      ],
      out_shape=[
          jax.ShapeDtypeStruct((EO, B), jnp.float32),
          jax.ShapeDtypeStruct((EU, B), jnp.float32),
      ],
  )(item_rows, xu2[:B], W_item)

  return (xut.T, xit.T)
